# baseline (speedup 1.0000x reference)
import numpy as np
import jax
import jax.numpy as jnp
from jax import lax
from jax.experimental import pallas as pl
from jax.experimental.pallas import tpu as pltpu
from jax.experimental.pallas import tpu_sc as plsc

B = 32
XS = 256
NPTS = 100000
NPAD = 102400
BLK = 2048
CHUNK = 10000
CLIP_MAX = np.float32(XS - 1.0 - 1e-4)


def _transform_body(lx, ly, lz, z, ct, r, s, px_o, py_o):
    zb = z[...]
    dn = (((1,), (1,)), ((), ()))
    dx = lax.dot_general(lx[...], zb, dn, preferred_element_type=jnp.float32)
    dy = lax.dot_general(ly[...], zb, dn, preferred_element_type=jnp.float32)
    dz = lax.dot_general(lz[...], zb, dn, preferred_element_type=jnp.float32)
    cx = dx + ct[0:1, :]
    cy = dy + ct[1:2, :]
    cz = dz + ct[2:3, :]
    rr = r[...]
    ss = s[...]
    crx = rr[:, 0:1] * cx + rr[:, 1:2] * cy + rr[:, 2:3] * cz + ss[:, 0:1] + 128.0
    cry = rr[:, 3:4] * cx + rr[:, 4:5] * cy + rr[:, 5:6] * cz + ss[:, 1:2] + 128.0
    px_o[...] = jnp.clip(crx, 0.0, CLIP_MAX)
    py_o[...] = jnp.clip(cry, 0.0, CLIP_MAX)


def _transform(latx, laty, latz, zp, ctp, rflat, shifts):
    return pl.pallas_call(
        _transform_body,
        grid=(NPAD // BLK,),
        in_specs=[
            pl.BlockSpec((B, 8), lambda j: (0, 0)),
            pl.BlockSpec((B, 8), lambda j: (0, 0)),
            pl.BlockSpec((B, 8), lambda j: (0, 0)),
            pl.BlockSpec((BLK, 8), lambda j: (j, 0)),
            pl.BlockSpec((8, BLK), lambda j: (0, j)),
            pl.BlockSpec((B, 9), lambda j: (0, 0)),
            pl.BlockSpec((B, 2), lambda j: (0, 0)),
        ],
        out_specs=[
            pl.BlockSpec((B, BLK), lambda j: (0, j)),
            pl.BlockSpec((B, BLK), lambda j: (0, j)),
        ],
        out_shape=[
            jax.ShapeDtypeStruct((B, NPAD), jnp.float32),
            jax.ShapeDtypeStruct((B, NPAD), jnp.float32),
        ],
    )(latx, laty, latz, zp, ctp, rflat, shifts)


_UNROLL = 25
_NCHUNKS = NPTS // CHUNK


def _sc_scatter_body(px_hbm, py_hbm, img_hbm, pxv, pyv, imgv, semx, semy):
    b = lax.axis_index("s") * 2 + lax.axis_index("c")
    base = b * NPAD
    zeros16 = jnp.zeros((16,), jnp.float32)

    def _issue(c, slot):
        pltpu.async_copy(px_hbm.at[pl.ds(base + c * CHUNK, CHUNK)],
                         pxv.at[pl.ds(slot * CHUNK, CHUNK)], semx)
        pltpu.async_copy(py_hbm.at[pl.ds(base + c * CHUNK, CHUNK)],
                         pyv.at[pl.ds(slot * CHUNK, CHUNK)], semy)

    _issue(0, 0)

    def zbody(i, _):
        for u in range(8):
            imgv[pl.ds(i * 128 + u * 16, 16)] = zeros16
        return _

    lax.fori_loop(0, (XS * XS) // 128, zbody, None)

    def chunk(c, _):
        slot = lax.rem(c, 2)
        soff = slot * CHUNK
        pltpu.make_async_copy(px_hbm.at[pl.ds(base + c * CHUNK, CHUNK)],
                              pxv.at[pl.ds(soff, CHUNK)], semx).wait()
        pltpu.make_async_copy(py_hbm.at[pl.ds(base + c * CHUNK, CHUNK)],
                              pyv.at[pl.ds(soff, CHUNK)], semy).wait()

        @pl.when(c + 1 < _NCHUNKS)
        def _():
            _issue(c + 1, 1 - slot)

        def pbody(i, _):
            ib = soff + i * (16 * _UNROLL)
            for u in range(_UNROLL):
                px = pxv[pl.ds(ib + u * 16, 16)]
                py = pyv[pl.ds(ib + u * 16, 16)]
                x0 = px.astype(jnp.int32)
                y0 = py.astype(jnp.int32)
                fx = px - x0.astype(jnp.float32)
                fy = py - y0.astype(jnp.float32)
                gx = 1.0 - fx
                gy = 1.0 - fy
                i00 = y0 * XS + x0
                plsc.addupdate_scatter(imgv, [i00], gx * gy)
                plsc.addupdate_scatter(imgv, [i00 + 1], fx * gy)
                plsc.addupdate_scatter(imgv, [i00 + XS], gx * fy)
                plsc.addupdate_scatter(imgv, [i00 + XS + 1], fx * fy)
            return _

        lax.fori_loop(0, CHUNK // (16 * _UNROLL), pbody, None)
        return _

    lax.fori_loop(0, _NCHUNKS, chunk, None)
    pltpu.sync_copy(imgv, img_hbm.at[pl.ds(b * (XS * XS), XS * XS)])


def _scatter(px_flat, py_flat):
    call = pl.kernel(
        _sc_scatter_body,
        out_type=jax.ShapeDtypeStruct((B * XS * XS,), jnp.float32),
        mesh=plsc.VectorSubcoreMesh(core_axis_name="c", subcore_axis_name="s"),
        scratch_types=[
            pltpu.VMEM((2 * CHUNK,), jnp.float32),
            pltpu.VMEM((2 * CHUNK,), jnp.float32),
            pltpu.VMEM((XS * XS,), jnp.float32),
            pltpu.SemaphoreType.DMA,
            pltpu.SemaphoreType.DMA,
        ],
        compiler_params=pltpu.CompilerParams(needs_layout_passes=False),
    )
    return call(px_flat, py_flat)


_n = np.arange(XS)
_ang = (2.0 * np.pi / XS) * np.outer(_n, _n)
_FR = np.cos(_ang).astype(np.float32)
_FI = (-np.sin(_ang)).astype(np.float32)
_GR = (np.cos(_ang) / XS).astype(np.float32)
_GI = (np.sin(_ang) / XS).astype(np.float32)


_DB = 4


def _dft_body(x_r, c_r, fr_r, fi_r, gr_r, gi_r, o_r):
    def dot(a, bb):
        return lax.dot_general(
            a, bb, (((1,), (0,)), ((), ())),
            preferred_element_type=jnp.float32)

    fr = fr_r[...]
    fi = fi_r[...]
    gr = gr_r[...]
    gi = gi_r[...]
    for i in range(_DB):
        x = x_r[i]
        tr = dot(x, fr)
        ti = dot(x, fi)
        ftr = dot(fr, tr) - dot(fi, ti)
        fti = dot(fr, ti) + dot(fi, tr)
        c = c_r[i]
        yr = ftr * c
        yi = fti * c
        ur = dot(yr, gr) - dot(yi, gi)
        ui = dot(yr, gi) + dot(yi, gr)
        o_r[i] = dot(gr, ur) - dot(gi, ui)


def _dft_filter(img, ctf_full):
    return pl.pallas_call(
        _dft_body,
        grid=(B // _DB,),
        in_specs=[
            pl.BlockSpec((_DB, XS, XS), lambda b: (b, 0, 0)),
            pl.BlockSpec((_DB, XS, XS), lambda b: (b, 0, 0)),
            pl.BlockSpec((XS, XS), lambda b: (0, 0)),
            pl.BlockSpec((XS, XS), lambda b: (0, 0)),
            pl.BlockSpec((XS, XS), lambda b: (0, 0)),
            pl.BlockSpec((XS, XS), lambda b: (0, 0)),
        ],
        out_specs=pl.BlockSpec((_DB, XS, XS), lambda b: (b, 0, 0)),
        out_shape=jax.ShapeDtypeStruct((B, XS, XS), jnp.float32),
    )(img, ctf_full, jnp.asarray(_FR), jnp.asarray(_FI), jnp.asarray(_GR), jnp.asarray(_GI))


def kernel(latent_x, latent_y, latent_z, Z, coords, R, shifts, ctf):
    zp = jnp.pad(Z, ((0, NPAD - NPTS), (0, 0)))
    ctp = jnp.pad(coords.T, ((0, 5), (0, NPAD - NPTS)))
    rflat = R.reshape(B, 9)
    px, py = _transform(latent_x, latent_y, latent_z, zp, ctp, rflat, shifts)
    img = _scatter(px.reshape(-1), py.reshape(-1)).reshape(B, XS, XS)
    t = jnp.flip(ctf[:, :, 1:128], axis=2)
    t = jnp.concatenate([t[:, 0:1, :], jnp.flip(t[:, 1:, :], axis=1)], axis=1)
    ctf_full = jnp.concatenate([ctf, t], axis=2)
    return _dft_filter(img, ctf_full)


# speedup vs baseline: 1.0049x; 1.0049x over previous
import numpy as np
import jax
import jax.numpy as jnp
from jax import lax
from jax.experimental import pallas as pl
from jax.experimental.pallas import tpu as pltpu
from jax.experimental.pallas import tpu_sc as plsc

B = 32
XS = 256
NPTS = 100000
NPAD = 102400
BLK = 2048
CHUNK = 10000
CLIP_MAX = np.float32(XS - 1.0 - 1e-4)


def _transform_body(lx, ly, lz, z, ct, r, s, px_o, py_o):
    zb = z[...]
    dn = (((1,), (1,)), ((), ()))
    dx = lax.dot_general(lx[...], zb, dn, preferred_element_type=jnp.float32)
    dy = lax.dot_general(ly[...], zb, dn, preferred_element_type=jnp.float32)
    dz = lax.dot_general(lz[...], zb, dn, preferred_element_type=jnp.float32)
    cx = dx + ct[0:1, :]
    cy = dy + ct[1:2, :]
    cz = dz + ct[2:3, :]
    rr = r[...]
    ss = s[...]
    crx = rr[:, 0:1] * cx + rr[:, 1:2] * cy + rr[:, 2:3] * cz + ss[:, 0:1] + 128.0
    cry = rr[:, 3:4] * cx + rr[:, 4:5] * cy + rr[:, 5:6] * cz + ss[:, 1:2] + 128.0
    px_o[...] = jnp.clip(crx, 0.0, CLIP_MAX)
    py_o[...] = jnp.clip(cry, 0.0, CLIP_MAX)


def _transform(latx, laty, latz, zp, ctp, rflat, shifts):
    return pl.pallas_call(
        _transform_body,
        grid=(NPAD // BLK,),
        in_specs=[
            pl.BlockSpec((B, 8), lambda j: (0, 0)),
            pl.BlockSpec((B, 8), lambda j: (0, 0)),
            pl.BlockSpec((B, 8), lambda j: (0, 0)),
            pl.BlockSpec((BLK, 8), lambda j: (j, 0)),
            pl.BlockSpec((8, BLK), lambda j: (0, j)),
            pl.BlockSpec((B, 9), lambda j: (0, 0)),
            pl.BlockSpec((B, 2), lambda j: (0, 0)),
        ],
        out_specs=[
            pl.BlockSpec((B, BLK), lambda j: (0, j)),
            pl.BlockSpec((B, BLK), lambda j: (0, j)),
        ],
        out_shape=[
            jax.ShapeDtypeStruct((B, NPAD), jnp.float32),
            jax.ShapeDtypeStruct((B, NPAD), jnp.float32),
        ],
    )(latx, laty, latz, zp, ctp, rflat, shifts)


_UNROLL = 25
_NCHUNKS = NPTS // CHUNK


def _sc_scatter_body(px_hbm, py_hbm, img_hbm, pxv, pyv, imgv, semx, semy):
    b = lax.axis_index("s") * 2 + lax.axis_index("c")
    base = b * NPAD
    zeros16 = jnp.zeros((16,), jnp.float32)

    def _issue(c, slot):
        pltpu.async_copy(px_hbm.at[pl.ds(base + c * CHUNK, CHUNK)],
                         pxv.at[pl.ds(slot * CHUNK, CHUNK)], semx)
        pltpu.async_copy(py_hbm.at[pl.ds(base + c * CHUNK, CHUNK)],
                         pyv.at[pl.ds(slot * CHUNK, CHUNK)], semy)

    _issue(0, 0)

    def zbody(i, _):
        for u in range(8):
            imgv[pl.ds(i * 128 + u * 16, 16)] = zeros16
        return _

    lax.fori_loop(0, (XS * XS) // 128, zbody, None)

    def chunk(c, _):
        slot = lax.rem(c, 2)
        soff = slot * CHUNK
        pltpu.make_async_copy(px_hbm.at[pl.ds(base + c * CHUNK, CHUNK)],
                              pxv.at[pl.ds(soff, CHUNK)], semx).wait()
        pltpu.make_async_copy(py_hbm.at[pl.ds(base + c * CHUNK, CHUNK)],
                              pyv.at[pl.ds(soff, CHUNK)], semy).wait()

        @pl.when(c + 1 < _NCHUNKS)
        def _():
            _issue(c + 1, 1 - slot)

        def pbody(i, _):
            ib = soff + i * (16 * _UNROLL)
            for u in range(_UNROLL):
                px = pxv[pl.ds(ib + u * 16, 16)]
                py = pyv[pl.ds(ib + u * 16, 16)]
                x0 = px.astype(jnp.int32)
                y0 = py.astype(jnp.int32)
                fx = px - x0.astype(jnp.float32)
                fy = py - y0.astype(jnp.float32)
                gx = 1.0 - fx
                gy = 1.0 - fy
                i00 = y0 * XS + x0
                plsc.addupdate_scatter(imgv, [i00], gx * gy)
                plsc.addupdate_scatter(imgv, [i00 + 1], fx * gy)
                plsc.addupdate_scatter(imgv, [i00 + XS], gx * fy)
                plsc.addupdate_scatter(imgv, [i00 + XS + 1], fx * fy)
            return _

        lax.fori_loop(0, CHUNK // (16 * _UNROLL), pbody, None)
        return _

    lax.fori_loop(0, _NCHUNKS, chunk, None)
    pltpu.sync_copy(imgv, img_hbm.at[pl.ds(b * (XS * XS), XS * XS)])


def _scatter(px_flat, py_flat):
    call = pl.kernel(
        _sc_scatter_body,
        out_type=jax.ShapeDtypeStruct((B * XS * XS,), jnp.float32),
        mesh=plsc.VectorSubcoreMesh(core_axis_name="c", subcore_axis_name="s"),
        scratch_types=[
            pltpu.VMEM((2 * CHUNK,), jnp.float32),
            pltpu.VMEM((2 * CHUNK,), jnp.float32),
            pltpu.VMEM((XS * XS,), jnp.float32),
            pltpu.SemaphoreType.DMA,
            pltpu.SemaphoreType.DMA,
        ],
        compiler_params=pltpu.CompilerParams(needs_layout_passes=False),
    )
    return call(px_flat, py_flat)


_n = np.arange(XS)
_H = XS // 2 + 1
_ang = (2.0 * np.pi / XS) * np.outer(_n, _n)
_FR = np.cos(_ang).astype(np.float32)
_FI = (-np.sin(_ang)).astype(np.float32)
_GR = (np.cos(_ang) / XS).astype(np.float32)
_GI = (np.sin(_ang) / XS).astype(np.float32)
_FRH = np.ascontiguousarray(_FR[:, :_H])
_FIH = np.ascontiguousarray(_FI[:, :_H])
_w = np.ones((_H, 1), np.float32)
_w[1:-1] = 2.0
_WVR = np.ascontiguousarray(_w * _GR[:_H, :])
_WVI = np.ascontiguousarray(_w * _GI[:_H, :])


_DB = 4


def _dft_body(x_r, c_r, frh_r, fih_r, fr_r, fi_r, gr_r, gi_r, wvr_r, wvi_r, o_r):
    def dot(a, bb):
        return lax.dot_general(
            a, bb, (((1,), (0,)), ((), ())),
            preferred_element_type=jnp.float32)

    frh = frh_r[...]
    fih = fih_r[...]
    fr = fr_r[...]
    fi = fi_r[...]
    gr = gr_r[...]
    gi = gi_r[...]
    wvr = wvr_r[...]
    wvi = wvi_r[...]
    for i in range(_DB):
        x = x_r[i]
        tr = dot(x, frh)
        ti = dot(x, fih)
        ftr = dot(fr, tr) - dot(fi, ti)
        fti = dot(fr, ti) + dot(fi, tr)
        c = c_r[i]
        yr = ftr * c
        yi = fti * c
        ur = dot(gr, yr) - dot(gi, yi)
        ui = dot(gr, yi) + dot(gi, yr)
        o_r[i] = dot(ur, wvr) - dot(ui, wvi)


def _dft_filter(img, ctf):
    full = lambda shape: (lambda b: tuple(0 for _ in shape))
    return pl.pallas_call(
        _dft_body,
        grid=(B // _DB,),
        in_specs=[
            pl.BlockSpec((_DB, XS, XS), lambda b: (b, 0, 0)),
            pl.BlockSpec((_DB, XS, _H), lambda b: (b, 0, 0)),
            pl.BlockSpec((XS, _H), lambda b: (0, 0)),
            pl.BlockSpec((XS, _H), lambda b: (0, 0)),
            pl.BlockSpec((XS, XS), lambda b: (0, 0)),
            pl.BlockSpec((XS, XS), lambda b: (0, 0)),
            pl.BlockSpec((XS, XS), lambda b: (0, 0)),
            pl.BlockSpec((XS, XS), lambda b: (0, 0)),
            pl.BlockSpec((_H, XS), lambda b: (0, 0)),
            pl.BlockSpec((_H, XS), lambda b: (0, 0)),
        ],
        out_specs=pl.BlockSpec((_DB, XS, XS), lambda b: (b, 0, 0)),
        out_shape=jax.ShapeDtypeStruct((B, XS, XS), jnp.float32),
    )(img, ctf, jnp.asarray(_FRH), jnp.asarray(_FIH), jnp.asarray(_FR),
      jnp.asarray(_FI), jnp.asarray(_GR), jnp.asarray(_GI),
      jnp.asarray(_WVR), jnp.asarray(_WVI))


def kernel(latent_x, latent_y, latent_z, Z, coords, R, shifts, ctf):
    zp = jnp.pad(Z, ((0, NPAD - NPTS), (0, 0)))
    ctp = jnp.pad(coords.T, ((0, 5), (0, NPAD - NPTS)))
    rflat = R.reshape(B, 9)
    px, py = _transform(latent_x, latent_y, latent_z, zp, ctp, rflat, shifts)
    img = _scatter(px.reshape(-1), py.reshape(-1)).reshape(B, XS, XS)
    return _dft_filter(img, ctf)


# speedup vs baseline: 1.0053x; 1.0004x over previous
import numpy as np
import jax
import jax.numpy as jnp
from jax import lax
from jax.experimental import pallas as pl
from jax.experimental.pallas import tpu as pltpu
from jax.experimental.pallas import tpu_sc as plsc

B = 32
XS = 256
NPTS = 100000
NPAD = 102400
BLK = 2048
CHUNK = 10000
CLIP_MAX = np.float32(XS - 1.0 - 1e-4)


def _transform_body(lx, ly, lz, z, ct, r, s, px_o, py_o):
    zb = z[...]
    dn = (((1,), (1,)), ((), ()))
    dx = lax.dot_general(lx[...], zb, dn, preferred_element_type=jnp.float32)
    dy = lax.dot_general(ly[...], zb, dn, preferred_element_type=jnp.float32)
    dz = lax.dot_general(lz[...], zb, dn, preferred_element_type=jnp.float32)
    cx = dx + ct[0:1, :]
    cy = dy + ct[1:2, :]
    cz = dz + ct[2:3, :]
    rr = r[...]
    ss = s[...]
    crx = rr[:, 0:1] * cx + rr[:, 1:2] * cy + rr[:, 2:3] * cz + ss[:, 0:1] + 128.0
    cry = rr[:, 3:4] * cx + rr[:, 4:5] * cy + rr[:, 5:6] * cz + ss[:, 1:2] + 128.0
    px_o[...] = jnp.clip(crx, 0.0, CLIP_MAX)
    py_o[...] = jnp.clip(cry, 0.0, CLIP_MAX)


def _transform(latx, laty, latz, zp, ctp, rflat, shifts):
    return pl.pallas_call(
        _transform_body,
        grid=(NPAD // BLK,),
        in_specs=[
            pl.BlockSpec((B, 8), lambda j: (0, 0)),
            pl.BlockSpec((B, 8), lambda j: (0, 0)),
            pl.BlockSpec((B, 8), lambda j: (0, 0)),
            pl.BlockSpec((BLK, 8), lambda j: (j, 0)),
            pl.BlockSpec((8, BLK), lambda j: (0, j)),
            pl.BlockSpec((B, 9), lambda j: (0, 0)),
            pl.BlockSpec((B, 2), lambda j: (0, 0)),
        ],
        out_specs=[
            pl.BlockSpec((B, BLK), lambda j: (0, j)),
            pl.BlockSpec((B, BLK), lambda j: (0, j)),
        ],
        out_shape=[
            jax.ShapeDtypeStruct((B, NPAD), jnp.float32),
            jax.ShapeDtypeStruct((B, NPAD), jnp.float32),
        ],
    )(latx, laty, latz, zp, ctp, rflat, shifts)


_UNROLL = 25
_NCHUNKS = NPTS // CHUNK


def _sc_scatter_body(px_hbm, py_hbm, img_hbm, pxv, pyv, imgv, semx, semy):
    b = lax.axis_index("s") * 2 + lax.axis_index("c")
    base = b * NPAD
    zeros16 = jnp.zeros((16,), jnp.float32)

    def _issue(c, slot):
        pltpu.async_copy(px_hbm.at[pl.ds(base + c * CHUNK, CHUNK)],
                         pxv.at[pl.ds(slot * CHUNK, CHUNK)], semx)
        pltpu.async_copy(py_hbm.at[pl.ds(base + c * CHUNK, CHUNK)],
                         pyv.at[pl.ds(slot * CHUNK, CHUNK)], semy)

    _issue(0, 0)

    def zbody(i, _):
        for u in range(8):
            imgv[pl.ds(i * 128 + u * 16, 16)] = zeros16
        return _

    lax.fori_loop(0, (XS * XS) // 128, zbody, None)

    def chunk(c, _):
        slot = lax.rem(c, 2)
        soff = slot * CHUNK
        pltpu.make_async_copy(px_hbm.at[pl.ds(base + c * CHUNK, CHUNK)],
                              pxv.at[pl.ds(soff, CHUNK)], semx).wait()
        pltpu.make_async_copy(py_hbm.at[pl.ds(base + c * CHUNK, CHUNK)],
                              pyv.at[pl.ds(soff, CHUNK)], semy).wait()

        @pl.when(c + 1 < _NCHUNKS)
        def _():
            _issue(c + 1, 1 - slot)

        def pbody(i, _):
            ib = soff + i * (16 * _UNROLL)
            for u in range(_UNROLL):
                px = pxv[pl.ds(ib + u * 16, 16)]
                py = pyv[pl.ds(ib + u * 16, 16)]
                x0 = px.astype(jnp.int32)
                y0 = py.astype(jnp.int32)
                fx = px - x0.astype(jnp.float32)
                fy = py - y0.astype(jnp.float32)
                gx = 1.0 - fx
                gy = 1.0 - fy
                i00 = y0 * XS + x0
                plsc.addupdate_scatter(imgv, [i00], gx * gy)
                plsc.addupdate_scatter(imgv, [i00 + 1], fx * gy)
                plsc.addupdate_scatter(imgv, [i00 + XS], gx * fy)
                plsc.addupdate_scatter(imgv, [i00 + XS + 1], fx * fy)
            return _

        lax.fori_loop(0, CHUNK // (16 * _UNROLL), pbody, None)
        return _

    lax.fori_loop(0, _NCHUNKS, chunk, None)
    pltpu.sync_copy(imgv, img_hbm.at[pl.ds(b * (XS * XS), XS * XS)])


def _scatter(px_flat, py_flat):
    call = pl.kernel(
        _sc_scatter_body,
        out_type=jax.ShapeDtypeStruct((B * XS * XS,), jnp.float32),
        mesh=plsc.VectorSubcoreMesh(core_axis_name="c", subcore_axis_name="s"),
        scratch_types=[
            pltpu.VMEM((2 * CHUNK,), jnp.float32),
            pltpu.VMEM((2 * CHUNK,), jnp.float32),
            pltpu.VMEM((XS * XS,), jnp.float32),
            pltpu.SemaphoreType.DMA,
            pltpu.SemaphoreType.DMA,
        ],
        compiler_params=pltpu.CompilerParams(needs_layout_passes=False),
    )
    return call(px_flat, py_flat)


_n = np.arange(XS)
_H = XS // 2 + 1
_ang = (2.0 * np.pi / XS) * np.outer(_n, _n)
_FR = np.cos(_ang).astype(np.float32)
_FI = (-np.sin(_ang)).astype(np.float32)
_GR = (np.cos(_ang) / XS).astype(np.float32)
_GI = (np.sin(_ang) / XS).astype(np.float32)
_FRH = np.ascontiguousarray(_FR[:, :_H])
_FIH = np.ascontiguousarray(_FI[:, :_H])
_w = np.ones((_H, 1), np.float32)
_w[1:-1] = 2.0
_WVR = np.ascontiguousarray(_w * _GR[:_H, :])
_WVI = np.ascontiguousarray(_w * _GI[:_H, :])


_DB = 8


def _dft_body(x_r, c_r, frh_r, fih_r, fr_r, fi_r, gr_r, gi_r, wvr_r, wvi_r, o_r):
    def dot(a, bb):
        return lax.dot_general(
            a, bb, (((1,), (0,)), ((), ())),
            preferred_element_type=jnp.float32)

    frh = frh_r[...]
    fih = fih_r[...]
    fr = fr_r[...]
    fi = fi_r[...]
    gr = gr_r[...]
    gi = gi_r[...]
    wvr = wvr_r[...]
    wvi = wvi_r[...]
    for i in range(_DB):
        x = x_r[i]
        tr = dot(x, frh)
        ti = dot(x, fih)
        ftr = dot(fr, tr) - dot(fi, ti)
        fti = dot(fr, ti) + dot(fi, tr)
        c = c_r[i]
        yr = ftr * c
        yi = fti * c
        ur = dot(gr, yr) - dot(gi, yi)
        ui = dot(gr, yi) + dot(gi, yr)
        o_r[i] = dot(ur, wvr) - dot(ui, wvi)


def _dft_filter(img, ctf):
    full = lambda shape: (lambda b: tuple(0 for _ in shape))
    return pl.pallas_call(
        _dft_body,
        grid=(B // _DB,),
        in_specs=[
            pl.BlockSpec((_DB, XS, XS), lambda b: (b, 0, 0)),
            pl.BlockSpec((_DB, XS, _H), lambda b: (b, 0, 0)),
            pl.BlockSpec((XS, _H), lambda b: (0, 0)),
            pl.BlockSpec((XS, _H), lambda b: (0, 0)),
            pl.BlockSpec((XS, XS), lambda b: (0, 0)),
            pl.BlockSpec((XS, XS), lambda b: (0, 0)),
            pl.BlockSpec((XS, XS), lambda b: (0, 0)),
            pl.BlockSpec((XS, XS), lambda b: (0, 0)),
            pl.BlockSpec((_H, XS), lambda b: (0, 0)),
            pl.BlockSpec((_H, XS), lambda b: (0, 0)),
        ],
        out_specs=pl.BlockSpec((_DB, XS, XS), lambda b: (b, 0, 0)),
        out_shape=jax.ShapeDtypeStruct((B, XS, XS), jnp.float32),
    )(img, ctf, jnp.asarray(_FRH), jnp.asarray(_FIH), jnp.asarray(_FR),
      jnp.asarray(_FI), jnp.asarray(_GR), jnp.asarray(_GI),
      jnp.asarray(_WVR), jnp.asarray(_WVI))


def kernel(latent_x, latent_y, latent_z, Z, coords, R, shifts, ctf):
    zp = jnp.pad(Z, ((0, NPAD - NPTS), (0, 0)))
    ctp = jnp.pad(coords.T, ((0, 5), (0, NPAD - NPTS)))
    rflat = R.reshape(B, 9)
    px, py = _transform(latent_x, latent_y, latent_z, zp, ctp, rflat, shifts)
    img = _scatter(px.reshape(-1), py.reshape(-1)).reshape(B, XS, XS)
    return _dft_filter(img, ctf)
